# speedup vs baseline: 1.6595x; 1.6595x over previous
import jax
import jax.numpy as jnp
from jax.experimental import pallas as pl
from jax.experimental.pallas import tpu as pltpu

_BLK_L = 512


def _add_kernel(feat_ref, pos_ref, out_ref):
    out_ref[...] = feat_ref[...] + pos_ref[...]


def kernel(feat, pos_table):
    B, L, D = feat.shape
    blk = _BLK_L
    grid = (L // blk, B)
    return pl.pallas_call(
        _add_kernel,
        grid=grid,
        in_specs=[
            pl.BlockSpec((1, blk, D), lambda l, b: (b, l, 0)),
            pl.BlockSpec((blk, D), lambda l, b: (l, 0)),
        ],
        out_specs=pl.BlockSpec((1, blk, D), lambda l, b: (b, l, 0)),
        out_shape=jax.ShapeDtypeStruct((B, L, D), feat.dtype),
        compiler_params=pltpu.CompilerParams(
            dimension_semantics=("arbitrary", "arbitrary"),
        ),
    )(feat, pos_table)


# speedup vs baseline: 1.7325x; 1.0440x over previous
import jax
import jax.numpy as jnp
from jax.experimental import pallas as pl
from jax.experimental.pallas import tpu as pltpu

_BLK_L = 1024


def _add_kernel(feat_ref, pos_ref, out_ref):
    out_ref[...] = feat_ref[...] + pos_ref[...]


def kernel(feat, pos_table):
    B, L, D = feat.shape
    blk = _BLK_L
    grid = (L // blk, B)
    return pl.pallas_call(
        _add_kernel,
        grid=grid,
        in_specs=[
            pl.BlockSpec((1, blk, D), lambda l, b: (b, l, 0)),
            pl.BlockSpec((blk, D), lambda l, b: (l, 0)),
        ],
        out_specs=pl.BlockSpec((1, blk, D), lambda l, b: (b, l, 0)),
        out_shape=jax.ShapeDtypeStruct((B, L, D), feat.dtype),
        compiler_params=pltpu.CompilerParams(
            dimension_semantics=("arbitrary", "arbitrary"),
        ),
    )(feat, pos_table)
